# baseline (speedup 1.0000x reference)
import functools

import jax
import jax.numpy as jnp
from jax import lax
from jax.experimental import pallas as pl
from jax.experimental.pallas import tpu as pltpu
from jax.experimental.pallas import tpu_sc as plsc

_L = 16


@functools.cache
def _make_kernel(B, D, NC, NS):
    NW = NC * NS
    chunk = B // NW
    mesh = plsc.VectorSubcoreMesh(core_axis_name="c", subcore_axis_name="s")

    @functools.partial(
        pl.kernel,
        mesh=mesh,
        out_type=jax.ShapeDtypeStruct((B,), jnp.float32),
        compiler_params=pltpu.CompilerParams(
            needs_layout_passes=False, use_tc_tiling_on_sc=False),
        scratch_types=[
            pltpu.VMEM((chunk,), jnp.int32),
            pltpu.VMEM((chunk,), jnp.int32),
            pltpu.VMEM((chunk,), jnp.int32),
            pltpu.VMEM((chunk, D), jnp.float32),
            pltpu.VMEM((chunk, D), jnp.float32),
            pltpu.VMEM((chunk, D), jnp.float32),
            pltpu.VMEM((chunk,), jnp.float32),
            pltpu.VMEM((chunk,), jnp.float32),
            pltpu.SemaphoreType.DMA,
        ],
    )
    def body(i_hbm, j_hbm, k_hbm, u_hbm, v_hbm, b_hbm, out_hbm,
             ii, jj, kk, u_rows, vj_rows, vk_rows, b_v, out_v, sem):
        wid = lax.axis_index("s") * NC + lax.axis_index("c")
        base = wid * chunk
        pltpu.sync_copy(i_hbm.at[pl.ds(base, chunk)], ii)
        pltpu.sync_copy(j_hbm.at[pl.ds(base, chunk)], jj)
        pltpu.sync_copy(k_hbm.at[pl.ds(base, chunk)], kk)
        c1 = pltpu.async_copy(u_hbm.at[ii], u_rows, sem)
        c2 = pltpu.async_copy(v_hbm.at[jj], vj_rows, sem)
        c3 = pltpu.async_copy(v_hbm.at[kk], vk_rows, sem)
        c4 = pltpu.async_copy(b_hbm.at[ii], b_v, sem)
        c1.wait()
        c2.wait()
        c3.wait()
        c4.wait()

        nchunks = D // _L
        lane = lax.iota(jnp.int32, _L)

        def group(g, carry):
            s = jnp.zeros((_L,), jnp.float32)
            for rr in range(_L):
                r = g * _L + rr
                t = jnp.zeros((_L,), jnp.float32)
                for c in range(nchunks):
                    sl = pl.ds(c * _L, _L)
                    t = t + u_rows[r, sl] * (vj_rows[r, sl] - vk_rows[r, sl])
                s = jnp.where(lane == rr, jnp.sum(t), s)
            x = s + b_v[pl.ds(g * _L, _L)]
            out_v[pl.ds(g * _L, _L)] = 1.0 / (1.0 + jnp.exp(-x))
            return carry

        lax.fori_loop(0, chunk // _L, group, 0)
        pltpu.sync_copy(out_v, out_hbm.at[pl.ds(base, chunk)])

    return body


def kernel(i, j, k, u_weight, v_weight, b_weight):
    B = i.shape[0]
    D = u_weight.shape[1]
    try:
        info = plsc.get_sparse_core_info()
        NC, NS = info.num_cores, info.num_subcores
    except Exception:
        NC, NS = 2, 16
    fn = _make_kernel(B, D, NC, NS)
    return fn(i.astype(jnp.int32), j.astype(jnp.int32), k.astype(jnp.int32),
              u_weight, v_weight, b_weight.reshape(-1))
